# baseline (speedup 1.0000x reference)
import functools

import jax
import jax.numpy as jnp
from jax import lax
from jax.experimental import pallas as pl
from jax.experimental.pallas import tpu as pltpu
from jax.experimental.pallas import tpu_sc as plsc

N = 10000
E = 320000
EP = 327680
NROW = EP // 128
ETR = NROW // 32
GB = 16
NA = 10112
ND = 1280
NASH = NA + ND
RW = 256
NAD = 10016


def _tc_a_body(x_ref, w_ref, ss_ref, sd_ref, tab_ref, adst_ref):
    h = jnp.dot(x_ref[...], w_ref[...], preferred_element_type=jnp.float32)
    asrc = jnp.dot(h, ss_ref[...], preferred_element_type=jnp.float32)
    adst = jnp.dot(h, sd_ref[...], preferred_element_type=jnp.float32)
    tab_ref[:, pl.ds(0, 128)] = h
    tab_ref[:, pl.ds(128, 16)] = asrc
    tab_ref[:, pl.ds(144, 112)] = jnp.zeros((N, 112), jnp.float32)
    adst_ref[pl.ds(0, N), pl.ds(0, 16)] = adst
    adst_ref[pl.ds(0, N), pl.ds(16, 112)] = jnp.zeros((N, 112), jnp.float32)
    adst_ref[pl.ds(N, NAD - N), :] = jnp.zeros((NAD - N, 128), jnp.float32)
    m = jnp.max(asrc, axis=0) + jnp.max(adst, axis=0)
    m = jnp.maximum(m, 0.2 * m)
    adst_ref[pl.ds(NAD - 8, 8), pl.ds(0, 16)] = jnp.broadcast_to(
        m[None, :], (8, 16))


def _tc_b_body(num_ref, den_ref, b1_ref, w2_ref, a2s_ref, a2d_ref, sel_ref,
               rep_ref, tab_ref, adst_ref):
    nums = num_ref[...]
    dens = den_ref[...]
    num = (nums[0] + nums[1])[:N]
    den = (dens[0] + dens[1])[:N]
    den128 = jnp.dot(den, sel_ref[...], preferred_element_type=jnp.float32)
    a = num / (den128 + 1e-16) + b1_ref[...]
    h2 = jnp.where(a > 0, a, jnp.exp(a) - 1.0)
    hh = jnp.dot(h2, w2_ref[...], preferred_element_type=jnp.float32)
    a2s = jnp.dot(hh, a2s_ref[...], preferred_element_type=jnp.float32)
    a2d = jnp.dot(hh, a2d_ref[...], preferred_element_type=jnp.float32)
    rep = rep_ref[...]
    tab_ref[:, pl.ds(0, 40)] = hh
    tab_ref[:, pl.ds(40, 88)] = jnp.zeros((N, 88), jnp.float32)
    tab_ref[:, pl.ds(128, 8)] = jnp.dot(a2s, rep)
    tab_ref[:, pl.ds(136, 120)] = jnp.zeros((N, 120), jnp.float32)
    adst_ref[pl.ds(0, N), pl.ds(0, 8)] = jnp.dot(a2d, rep)
    adst_ref[pl.ds(0, N), pl.ds(8, 120)] = jnp.zeros((N, 120), jnp.float32)
    adst_ref[pl.ds(N, NAD - N), :] = jnp.zeros((NAD - N, 128), jnp.float32)
    t = jnp.max(a2s) + jnp.max(a2d)
    m2 = jnp.maximum(t, 0.2 * t)
    adst_ref[pl.ds(NAD - 8, 8), pl.ds(0, 16)] = jnp.full(
        (8, 16), m2, jnp.float32)


def _tc_c_body(num_ref, den_ref, b2_ref, out_ref):
    nums = num_ref[...]
    dens = den_ref[...]
    num = (nums[0] + nums[1])[:N, :40]
    den = (dens[0] + dens[1])[:N, :1]
    logits = num / (den + 1e-16) + b2_ref[...]
    mx = jnp.max(logits, axis=1, keepdims=True)
    s = logits - mx
    lse = jnp.log(jnp.sum(jnp.exp(s), axis=1, keepdims=True))
    out_ref[...] = s - lse


def _sc_body(nheads,
             tab_hbm, adst_hbm, eir_hbm, out_hbm,
             idq, rowbuf, adb, sd, isrc, idst, idst48, sh,
             semA, semB):
    c = lax.axis_index("c")
    s = lax.axis_index("s")
    wid = s * 2 + c

    def zrow(i, _):
        for q in range(8):
            sd[0, i, pl.ds(16 * q, 16)] = jnp.zeros((16,), jnp.float32)
            sd[1, i, pl.ds(16 * q, 16)] = jnp.zeros((16,), jnp.float32)
        return 0

    lax.fori_loop(0, GB, zrow, 0)

    def zn(i, _):
        pltpu.sync_copy(sd.at[0, pl.ds(0, 8)],
                        sh.at[pl.ds(s * 632 + i * 8, 8)])
        return 0

    lax.fori_loop(0, 79, zn, 0)

    def zd(i, _):
        pltpu.sync_copy(sd.at[0, pl.ds(0, 8)],
                        sh.at[pl.ds(NA + s * 80 + i * 8, 8)])
        return 0

    lax.fori_loop(0, 10, zd, 0)

    pltpu.sync_copy(adst_hbm.at[pl.ds(NAD - 8, 8)], adb.at[0, pl.ds(0, 8)])
    mv = adb[0, 0, pl.ds(0, 16)]
    plsc.subcore_barrier()

    def block(blk, _):
        pltpu.sync_copy(eir_hbm.at[0, pl.ds(wid * ETR + blk * 8, 8)],
                        idq.at[0])
        pltpu.sync_copy(eir_hbm.at[1, pl.ds(wid * ETR + blk * 8, 8)],
                        idq.at[1])

        def row(i, _):
            def pair(p, _):
                cps = []
                for par in range(2):
                    hq = p * 2 + par
                    v = idq[0, i, pl.ds(hq * GB, 16)]
                    isrc[par, pl.ds(0, 16)] = v
                    w = idq[1, i, pl.ds(hq * GB, 16)]
                    idst[par, pl.ds(0, 16)] = w
                    idst48[par, pl.ds(0, 16)] = w
                    sem = semA if par == 0 else semB
                    c1 = pltpu.make_async_copy(
                        tab_hbm.at[isrc.at[par]], rowbuf.at[par], sem)
                    c2 = pltpu.make_async_copy(
                        adst_hbm.at[idst.at[par]], adb.at[par], sem)
                    c1.start()
                    c2.start()
                    cps.append((c1, c2))

                for par in range(2):
                    c1, c2 = cps[par]
                    c1.wait()
                    c2.wait()

                    def edge(j, _):
                        d = idst48[par, pl.ds(j, 16)][0]
                        g = d & 7
                        asr = rowbuf[par, j, pl.ds(128, 16)]
                        adv = adb[par, j, pl.ds(0, 16)]
                        e = asr + adv
                        f = jnp.exp(jnp.maximum(e, 0.2 * e) - mv)
                        for q in range(8):
                            sd[1, j, pl.ds(16 * q, 16)] = jnp.where(
                                g == q, f, jnp.zeros((16,), jnp.float32))
                        if nheads > 1:
                            for q in range(8):
                                sd[0, j, pl.ds(16 * q, 16)] = (
                                    f[q] * rowbuf[par, j, pl.ds(16 * q, 16)])
                        else:
                            fs = f[0]
                            for q in range(3):
                                sd[0, j, pl.ds(16 * q, 16)] = (
                                    fs * rowbuf[par, j, pl.ds(16 * q, 16)])
                        return 0

                    lax.fori_loop(0, GB, edge, 0)

                    v = idst48[par, pl.ds(0, 16)]
                    isrc[par, pl.ds(0, 16)] = jnp.right_shift(v, 3) + NA
                    pltpu.sync_copy(sd.at[0], sh.at[idst.at[par]], add=True)
                    pltpu.sync_copy(sd.at[1], sh.at[isrc.at[par]], add=True)
                return 0

            lax.fori_loop(0, 4, pair, 0)
            return 0

        lax.fori_loop(0, 8, row, 0)
        return 0

    lax.fori_loop(0, ETR // 8, block, 0)
    plsc.subcore_barrier()
    pltpu.sync_copy(sh.at[pl.ds(s * 632, 632)],
                    out_hbm.at[c, pl.ds(s * 632, 632)])
    pltpu.sync_copy(sh.at[pl.ds(NA + s * 80, 80)],
                    out_hbm.at[c, pl.ds(NA + s * 80, 80)])


def _make_sc(nheads):
    return functools.partial(
        pl.kernel,
        functools.partial(_sc_body, nheads),
        out_type=[
            jax.ShapeDtypeStruct((2, NASH, 128), jnp.float32),
        ],
        mesh=plsc.VectorSubcoreMesh(core_axis_name="c", subcore_axis_name="s"),
        scratch_types=[
            pltpu.VMEM((2, 8, 128), jnp.int32),
            pltpu.VMEM((2, GB, RW), jnp.float32),
            pltpu.VMEM((2, GB, 128), jnp.float32),
            pltpu.VMEM((2, GB, 128), jnp.float32),
            pltpu.VMEM((2, GB), jnp.int32),
            pltpu.VMEM((2, GB), jnp.int32),
            pltpu.VMEM((2, GB + 16), jnp.int32),
            pltpu.VMEM_SHARED((NASH, 128), jnp.float32),
            pltpu.SemaphoreType.DMA,
            pltpu.SemaphoreType.DMA,
        ],
    )()


def kernel(x, edge_index, W1, att_src1, att_dst1, b1, W2, att_src2, att_dst2, b2):
    pad_src = jnp.zeros((1, EP - E), jnp.int32)
    pad_dst = jnp.full((1, EP - E), N, jnp.int32)
    eir = jnp.concatenate(
        [edge_index, jnp.concatenate([pad_src, pad_dst], axis=0)],
        axis=1).reshape(2, NROW, 128)

    rows = jnp.arange(128)
    cols = jnp.arange(16)
    onehot = (cols[None, :] == (rows // 16)[:, None]).astype(jnp.float32)
    ss1 = onehot * att_src1.reshape(-1)[:, None]
    sd1 = onehot * att_dst1.reshape(-1)[:, None]
    sel = onehot.T

    tab1, adst1 = pl.pallas_call(
        _tc_a_body,
        out_shape=[
            jax.ShapeDtypeStruct((N, RW), jnp.float32),
            jax.ShapeDtypeStruct((NAD, 128), jnp.float32),
        ],
    )(x, W1, ss1, sd1)

    acc1, = _make_sc(8)(tab1, adst1, eir)
    num1 = lax.slice_in_dim(acc1, 0, NA, axis=1)
    den1 = lax.slice_in_dim(acc1, NA, NASH, axis=1).reshape(2, ND * 8, 16)

    tab2, adst2 = pl.pallas_call(
        _tc_b_body,
        out_shape=[
            jax.ShapeDtypeStruct((N, RW), jnp.float32),
            jax.ShapeDtypeStruct((NAD, 128), jnp.float32),
        ],
    )(num1, den1, b1.reshape(1, 128), W2, att_src2.reshape(40, 1),
      att_dst2.reshape(40, 1), sel, jnp.ones((1, 8), jnp.float32))

    acc2, = _make_sc(1)(tab2, adst2, eir)
    num2 = lax.slice_in_dim(acc2, 0, NA, axis=1)
    den2 = lax.slice_in_dim(acc2, NA, NASH, axis=1).reshape(2, ND * 8, 16)

    out = pl.pallas_call(
        _tc_c_body,
        out_shape=jax.ShapeDtypeStruct((N, 40), jnp.float32),
    )(num2, den2, b2.reshape(1, 40))
    return out


# speedup vs baseline: 1.5620x; 1.5620x over previous
import functools

import jax
import jax.numpy as jnp
from jax import lax
from jax.experimental import pallas as pl
from jax.experimental.pallas import tpu as pltpu
from jax.experimental.pallas import tpu_sc as plsc

N = 10000
E = 320000
EP = 327680
NROW = EP // 128
ETR = NROW // 32
GB = 16
NA = 10112
ND = 1280
NASH = NA + ND
RW = 256
NAD = 10016


def _tc_a_body(x_ref, w_ref, ss_ref, sd_ref, tab_ref, adst_ref):
    h = jnp.dot(x_ref[...], w_ref[...], preferred_element_type=jnp.float32)
    asrc = jnp.dot(h, ss_ref[...], preferred_element_type=jnp.float32)
    adst = jnp.dot(h, sd_ref[...], preferred_element_type=jnp.float32)
    tab_ref[:, pl.ds(0, 128)] = h
    tab_ref[:, pl.ds(128, 16)] = asrc
    tab_ref[:, pl.ds(144, 112)] = jnp.zeros((N, 112), jnp.float32)
    adst_ref[pl.ds(0, N), pl.ds(0, 16)] = adst
    adst_ref[pl.ds(0, N), pl.ds(16, 112)] = jnp.zeros((N, 112), jnp.float32)
    adst_ref[pl.ds(N, NAD - N), :] = jnp.zeros((NAD - N, 128), jnp.float32)
    m = jnp.max(asrc, axis=0) + jnp.max(adst, axis=0)
    m = jnp.maximum(m, 0.2 * m)
    adst_ref[pl.ds(NAD - 8, 8), pl.ds(0, 16)] = jnp.broadcast_to(
        m[None, :], (8, 16))


def _tc_b_body(num_ref, den_ref, b1_ref, w2_ref, a2s_ref, a2d_ref, sel_ref,
               rep_ref, tab_ref, adst_ref):
    nums = num_ref[...]
    dens = den_ref[...]
    num = (nums[0] + nums[1])[:N]
    den = (dens[0] + dens[1])[:N]
    den128 = jnp.dot(den, sel_ref[...], preferred_element_type=jnp.float32)
    a = num / (den128 + 1e-16) + b1_ref[...]
    h2 = jnp.where(a > 0, a, jnp.exp(a) - 1.0)
    hh = jnp.dot(h2, w2_ref[...], preferred_element_type=jnp.float32)
    a2s = jnp.dot(hh, a2s_ref[...], preferred_element_type=jnp.float32)
    a2d = jnp.dot(hh, a2d_ref[...], preferred_element_type=jnp.float32)
    rep = rep_ref[...]
    tab_ref[:, pl.ds(0, 40)] = hh
    tab_ref[:, pl.ds(40, 1)] = jnp.ones((N, 1), jnp.float32)
    tab_ref[:, pl.ds(41, 7)] = jnp.zeros((N, 7), jnp.float32)
    tab_ref[:, pl.ds(48, 8)] = jnp.dot(a2s, rep)
    tab_ref[:, pl.ds(56, 72)] = jnp.zeros((N, 72), jnp.float32)
    adst_ref[pl.ds(0, N), pl.ds(0, 8)] = jnp.dot(a2d, rep)
    adst_ref[pl.ds(0, N), pl.ds(8, 120)] = jnp.zeros((N, 120), jnp.float32)
    adst_ref[pl.ds(N, NAD - N), :] = jnp.zeros((NAD - N, 128), jnp.float32)
    t = jnp.max(a2s) + jnp.max(a2d)
    m2 = jnp.maximum(t, 0.2 * t)
    adst_ref[pl.ds(NAD - 8, 8), pl.ds(0, 16)] = jnp.full(
        (8, 16), m2, jnp.float32)


def _tc_c_body(num_ref, b2_ref, out_ref):
    nums = num_ref[...]
    numf = (nums[0] + nums[1])[:N]
    num = numf[:, :40]
    den = numf[:, 40:41]
    logits = num / (den + 1e-16) + b2_ref[...]
    mx = jnp.max(logits, axis=1, keepdims=True)
    s = logits - mx
    lse = jnp.log(jnp.sum(jnp.exp(s), axis=1, keepdims=True))
    out_ref[...] = s - lse


def _sc_body(nheads,
             tab_hbm, adst_hbm, eir_hbm, out_hbm,
             idq, rowbuf, adb, sd, isrc, idst, idst48, sh,
             semA, semB):
    c = lax.axis_index("c")
    s = lax.axis_index("s")
    wid = s * 2 + c

    def zrow(i, _):
        for t in range(4):
            for q in range(8):
                sd[t, i, pl.ds(16 * q, 16)] = jnp.zeros((16,), jnp.float32)
        return 0

    lax.fori_loop(0, GB, zrow, 0)

    def zn(i, _):
        pltpu.sync_copy(sd.at[0, pl.ds(0, 8)],
                        sh.at[pl.ds(s * 632 + i * 8, 8)])
        return 0

    lax.fori_loop(0, 79, zn, 0)

    def zd(i, _):
        pltpu.sync_copy(sd.at[0, pl.ds(0, 8)],
                        sh.at[pl.ds(NA + s * 80 + i * 8, 8)])
        return 0

    lax.fori_loop(0, 10, zd, 0)

    pltpu.sync_copy(adst_hbm.at[pl.ds(NAD - 8, 8)], adb.at[0, pl.ds(0, 8)])
    mv = adb[0, 0, pl.ds(0, 16)]
    plsc.subcore_barrier()

    def block(blk, _):
        pltpu.sync_copy(eir_hbm.at[0, pl.ds(wid * ETR + blk * 8, 8)],
                        idq.at[0])
        pltpu.sync_copy(eir_hbm.at[1, pl.ds(wid * ETR + blk * 8, 8)],
                        idq.at[1])

        def row(i, _):
            def pair(p, _):
                cps = []
                for par in range(2):
                    hq = p * 2 + par
                    v = idq[0, i, pl.ds(hq * GB, 16)]
                    isrc[par, pl.ds(0, 16)] = v
                    w = idq[1, i, pl.ds(hq * GB, 16)]
                    idst[par, pl.ds(0, 16)] = w
                    idst48[par, pl.ds(0, 16)] = w
                    sem = semA if par == 0 else semB
                    c1 = pltpu.make_async_copy(
                        tab_hbm.at[isrc.at[par]], rowbuf.at[par], sem)
                    c2 = pltpu.make_async_copy(
                        adst_hbm.at[idst.at[par]], adb.at[par], sem)
                    c1.start()
                    c2.start()
                    cps.append((c1, c2))

                prev = []
                for par in range(2):
                    c1, c2 = cps[par]
                    c1.wait()
                    c2.wait()
                    sk = 2 * par

                    aoff = 128 if nheads > 1 else 48
                    for j in range(GB):
                        asr = rowbuf[par, j, pl.ds(aoff, 16)]
                        adv = adb[par, j, pl.ds(0, 16)]
                        e = asr + adv
                        f = jnp.exp(jnp.maximum(e, 0.2 * e) - mv)
                        if nheads > 1:
                            d = idst48[par, pl.ds(j, 16)][0]
                            g = d & 7
                            for q in range(8):
                                sd[sk + 1, j, pl.ds(16 * q, 16)] = jnp.where(
                                    g == q, f, jnp.zeros((16,), jnp.float32))
                            for q in range(8):
                                sd[sk, j, pl.ds(16 * q, 16)] = (
                                    f[q] * rowbuf[par, j, pl.ds(16 * q, 16)])
                        else:
                            fs = f[0]
                            for q in range(3):
                                sd[sk, j, pl.ds(16 * q, 16)] = (
                                    fs * rowbuf[par, j, pl.ds(16 * q, 16)])

                    for h in prev:
                        h.wait()
                    prev = []
                    sem = semA if par == 0 else semB
                    prev.append(pltpu.async_copy(
                        sd.at[sk], sh.at[idst.at[par]], sem, add=True))
                    if nheads > 1:
                        v = idst48[par, pl.ds(0, 16)]
                        isrc[par, pl.ds(0, 16)] = (
                            jnp.right_shift(v, 3) + NA)
                        prev.append(pltpu.async_copy(
                            sd.at[sk + 1], sh.at[isrc.at[par]], sem,
                            add=True))
                for h in prev:
                    h.wait()
                return 0

            lax.fori_loop(0, 4, pair, 0)
            return 0

        lax.fori_loop(0, 8, row, 0)
        return 0

    lax.fori_loop(0, ETR // 8, block, 0)
    plsc.subcore_barrier()
    pltpu.sync_copy(sh.at[pl.ds(s * 632, 632)],
                    out_hbm.at[c, pl.ds(s * 632, 632)])
    pltpu.sync_copy(sh.at[pl.ds(NA + s * 80, 80)],
                    out_hbm.at[c, pl.ds(NA + s * 80, 80)])


def _make_sc(nheads):
    rw = RW if nheads > 1 else 128
    return functools.partial(
        pl.kernel,
        functools.partial(_sc_body, nheads),
        out_type=[
            jax.ShapeDtypeStruct((2, NASH, 128), jnp.float32),
        ],
        mesh=plsc.VectorSubcoreMesh(core_axis_name="c", subcore_axis_name="s"),
        scratch_types=[
            pltpu.VMEM((2, 8, 128), jnp.int32),
            pltpu.VMEM((2, GB, rw), jnp.float32),
            pltpu.VMEM((2, GB, 128), jnp.float32),
            pltpu.VMEM((4, GB, 128), jnp.float32),
            pltpu.VMEM((2, GB), jnp.int32),
            pltpu.VMEM((2, GB), jnp.int32),
            pltpu.VMEM((2, GB + 16), jnp.int32),
            pltpu.VMEM_SHARED((NASH, 128), jnp.float32),
            pltpu.SemaphoreType.DMA,
            pltpu.SemaphoreType.DMA,
        ],
    )()


def kernel(x, edge_index, W1, att_src1, att_dst1, b1, W2, att_src2, att_dst2, b2):
    pad_src = jnp.zeros((1, EP - E), jnp.int32)
    pad_dst = jnp.full((1, EP - E), N, jnp.int32)
    eir = jnp.concatenate(
        [edge_index, jnp.concatenate([pad_src, pad_dst], axis=0)],
        axis=1).reshape(2, NROW, 128)

    rows = jnp.arange(128)
    cols = jnp.arange(16)
    onehot = (cols[None, :] == (rows // 16)[:, None]).astype(jnp.float32)
    ss1 = onehot * att_src1.reshape(-1)[:, None]
    sd1 = onehot * att_dst1.reshape(-1)[:, None]
    sel = onehot.T

    tab1, adst1 = pl.pallas_call(
        _tc_a_body,
        out_shape=[
            jax.ShapeDtypeStruct((N, RW), jnp.float32),
            jax.ShapeDtypeStruct((NAD, 128), jnp.float32),
        ],
    )(x, W1, ss1, sd1)

    acc1, = _make_sc(8)(tab1, adst1, eir)
    num1 = lax.slice_in_dim(acc1, 0, NA, axis=1)
    den1 = lax.slice_in_dim(acc1, NA, NASH, axis=1).reshape(2, ND * 8, 16)

    tab2, adst2 = pl.pallas_call(
        _tc_b_body,
        out_shape=[
            jax.ShapeDtypeStruct((N, 128), jnp.float32),
            jax.ShapeDtypeStruct((NAD, 128), jnp.float32),
        ],
    )(num1, den1, b1.reshape(1, 128), W2, att_src2.reshape(40, 1),
      att_dst2.reshape(40, 1), sel, jnp.ones((1, 8), jnp.float32))

    acc2, = _make_sc(1)(tab2, adst2, eir)
    num2 = lax.slice_in_dim(acc2, 0, NA, axis=1)

    out = pl.pallas_call(
        _tc_c_body,
        out_shape=jax.ShapeDtypeStruct((N, 40), jnp.float32),
    )(num2, b2.reshape(1, 40))
    return out
